# baseline (speedup 1.0000x reference)
import functools

import jax
import jax.numpy as jnp
from jax import lax
from jax.experimental import pallas as pl
from jax.experimental.pallas import tpu as pltpu
from jax.experimental.pallas import tpu_sc as plsc

_N = 16384
_OUT_D = 129
_PAD_D = 128
_NC = 2
_NS = 16
_NW = _NC * _NS
_RPW = _N // _NW
_CHUNK = 128
_NCHUNK = _RPW // _CHUNK
_N_TYPES = 16
_XCOLS = 8


def _body(x_hbm, tbl_hbm, out_hbm, x8_v, idx_v, pre_v, zcol_v, rows_v, sem):
    wid = lax.axis_index("s") * _NC + lax.axis_index("c")
    base = wid * _RPW

    pltpu.sync_copy(x_hbm.at[pl.ds(base, _RPW), pl.ds(0, _XCOLS)], x8_v)

    lanes = lax.iota(jnp.int32, 16)
    col0 = jnp.zeros((16,), jnp.int32)
    col1 = jnp.ones((16,), jnp.int32)
    zero16 = jnp.zeros((16,), jnp.int32)
    cap16 = jnp.full((16,), _N_TYPES - 1, jnp.int32)

    for i in range(_RPW // 16):
        rows = lanes + i * 16
        x1 = plsc.load_gather(x8_v, [rows, col1])
        idx = jnp.minimum(jnp.maximum(x1.astype(jnp.int32), zero16), cap16)
        idx_row = idx_v.at[i // 8]
        idx_row[pl.ds((i % 8) * 16, 16)] = idx
        pre = plsc.load_gather(x8_v, [rows, col0])
        plsc.store_scatter(pre_v, [rows, col0], pre)

    cps = []
    for j in range(_NCHUNK):
        cps.append(
            pltpu.async_copy(
                tbl_hbm.at[idx_v.at[j]],
                rows_v.at[pl.ds(j * _CHUNK, _CHUNK)],
                sem,
            )
        )
    zf16 = jnp.zeros((16,), jnp.float32)
    for i in range(_RPW // 16):
        plsc.store_scatter(zcol_v, [lanes + i * 16, col0], zf16)
    for cp in cps:
        cp.wait()

    for i in range(_RPW // 16):
        rows = lanes + i * 16
        pre = plsc.load_gather(pre_v, [rows, col0])
        plsc.store_scatter(rows_v, [rows, col0], pre)

    pltpu.sync_copy(rows_v, out_hbm.at[pl.ds(base, _RPW), pl.ds(0, _PAD_D)])
    pltpu.sync_copy(zcol_v, out_hbm.at[pl.ds(base, _RPW), pl.ds(_PAD_D, 1)])


@jax.jit
def _sc_embed(X, tbl_padded):
    mesh = plsc.VectorSubcoreMesh(core_axis_name="c", subcore_axis_name="s")
    return pl.kernel(
        _body,
        out_type=jax.ShapeDtypeStruct((_N, _OUT_D), jnp.float32),
        mesh=mesh,
        scratch_types=[
            pltpu.VMEM((_RPW, _XCOLS), jnp.float32),
            pltpu.VMEM((_NCHUNK, _CHUNK), jnp.int32),
            pltpu.VMEM((_RPW, 1), jnp.float32),
            pltpu.VMEM((_RPW, 1), jnp.float32),
            pltpu.VMEM((_RPW, _PAD_D), jnp.float32),
            pltpu.SemaphoreType.DMA,
        ],
        compiler_params=pltpu.CompilerParams(
            use_tc_tiling_on_sc=False, needs_layout_passes=False
        ),
    )(X, tbl_padded)


def kernel(X, type_table):
    n_types, emb_d = type_table.shape
    tbl_padded = (
        jnp.zeros((n_types, _PAD_D), jnp.float32)
        .at[:, 1 : 1 + emb_d]
        .set(type_table)
    )
    return _sc_embed(X, tbl_padded)


# speedup vs baseline: 6.1039x; 6.1039x over previous
import jax
import jax.numpy as jnp
from jax import lax
from jax.experimental import pallas as pl
from jax.experimental.pallas import tpu as pltpu
from jax.experimental.pallas import tpu_sc as plsc

_N = 16384
_IN_D = 130
_OUT_D = 129
_EMB_D = 64
_NC = 2
_NS = 16
_NW = _NC * _NS
_RPW = _N // _NW
_CHUNK = 128
_NCHUNK = _RPW // _CHUNK
_N_TYPES = 16


def _body(x_hbm, tbl_hbm, out_hbm, tbl_v, xb0, xb1, ob0, ob1,
          sx0, sx1, so0, so1):
    wid = lax.axis_index("s") * _NC + lax.axis_index("c")
    base = wid * _RPW

    pltpu.sync_copy(tbl_hbm, tbl_v)

    lanes = lax.iota(jnp.int32, 16)
    col0 = jnp.zeros((16,), jnp.int32)
    col1 = jnp.ones((16,), jnp.int32)
    zero16 = jnp.zeros((16,), jnp.int32)
    cap16 = jnp.full((16,), _N_TYPES - 1, jnp.int32)
    zf16 = jnp.zeros((16,), jnp.float32)

    xbufs, obufs = (xb0, xb1), (ob0, ob1)
    xsems, osems = (sx0, sx1), (so0, so1)

    def assemble(xbuf, obuf):
        def group(g, _):
            rows = lanes + g * 16
            x1 = plsc.load_gather(xbuf, [rows, col1])
            idx = jnp.minimum(jnp.maximum(x1.astype(jnp.int32), zero16), cap16)
            pre = plsc.load_gather(xbuf, [rows, col0])
            plsc.store_scatter(obuf, [rows, col0], pre)
            for c in range(1, 1 + _EMB_D):
                v = plsc.load_gather(tbl_v, [idx, jnp.full((16,), c - 1, jnp.int32)])
                plsc.store_scatter(obuf, [rows, jnp.full((16,), c, jnp.int32)], v)
            for c in range(1 + _EMB_D, _OUT_D):
                plsc.store_scatter(obuf, [rows, jnp.full((16,), c, jnp.int32)], zf16)
            return 0
        lax.fori_loop(0, _CHUNK // 16, group, 0)

    cpx = [None] * _NCHUNK
    cpo = [None] * _NCHUNK
    cpx[0] = pltpu.async_copy(x_hbm.at[pl.ds(base, _CHUNK)], xbufs[0], xsems[0])
    for j in range(_NCHUNK):
        if j + 1 < _NCHUNK:
            cpx[j + 1] = pltpu.async_copy(
                x_hbm.at[pl.ds(base + (j + 1) * _CHUNK, _CHUNK)],
                xbufs[(j + 1) % 2],
                xsems[(j + 1) % 2],
            )
        cpx[j].wait()
        if j >= 2:
            cpo[j - 2].wait()
        assemble(xbufs[j % 2], obufs[j % 2])
        cpo[j] = pltpu.async_copy(
            obufs[j % 2],
            out_hbm.at[pl.ds(base + j * _CHUNK, _CHUNK)],
            osems[j % 2],
        )
    cpo[_NCHUNK - 2].wait()
    cpo[_NCHUNK - 1].wait()


@jax.jit
def _sc_embed(X, type_table):
    mesh = plsc.VectorSubcoreMesh(core_axis_name="c", subcore_axis_name="s")
    return pl.kernel(
        _body,
        out_type=jax.ShapeDtypeStruct((_N, _OUT_D), jnp.float32),
        mesh=mesh,
        scratch_types=[
            pltpu.VMEM((_N_TYPES, _EMB_D), jnp.float32),
            pltpu.VMEM((_CHUNK, _IN_D), jnp.float32),
            pltpu.VMEM((_CHUNK, _IN_D), jnp.float32),
            pltpu.VMEM((_CHUNK, _OUT_D), jnp.float32),
            pltpu.VMEM((_CHUNK, _OUT_D), jnp.float32),
            pltpu.SemaphoreType.DMA,
            pltpu.SemaphoreType.DMA,
            pltpu.SemaphoreType.DMA,
            pltpu.SemaphoreType.DMA,
        ],
        compiler_params=pltpu.CompilerParams(
            use_tc_tiling_on_sc=False, needs_layout_passes=False
        ),
    )(X, type_table)


def kernel(X, type_table):
    return _sc_embed(X, type_table)
